# baseline (speedup 1.0000x reference)
import jax
import jax.numpy as jnp
from jax.experimental import pallas as pl
from jax.experimental.pallas import tpu as pltpu

_N_STRIPES = 4


def _dma_copy_kernel(x_ref, o_ref, sems):
    rows = x_ref.shape[0] // _N_STRIPES
    for i in range(_N_STRIPES):
        pltpu.make_async_copy(
            x_ref.at[pl.ds(i * rows, rows), :],
            o_ref.at[pl.ds(i * rows, rows), :],
            sems.at[i],
        ).start()
    for i in range(_N_STRIPES):
        pltpu.make_async_copy(
            x_ref.at[pl.ds(i * rows, rows), :],
            o_ref.at[pl.ds(i * rows, rows), :],
            sems.at[i],
        ).wait()


def kernel(x, scale_p, scale_n):
    del scale_p, scale_n
    m, n = x.shape
    out = pl.pallas_call(
        _dma_copy_kernel,
        in_specs=[pl.BlockSpec(memory_space=pl.ANY)],
        out_specs=pl.BlockSpec(memory_space=pl.ANY),
        out_shape=jax.ShapeDtypeStruct((m, n), x.dtype),
        scratch_shapes=[pltpu.SemaphoreType.DMA((_N_STRIPES,))],
    )(x)
    return out


# speedup vs baseline: 29.4033x; 29.4033x over previous
import jax
import jax.numpy as jnp
from jax.experimental import pallas as pl
from jax.experimental.pallas import tpu as pltpu


def _copy_kernel(x_ref, o_ref):
    o_ref[...] = x_ref[...]


def kernel(x, scale_p, scale_n):
    del scale_p, scale_n
    m, n = x.shape
    blk = 8
    out = pl.pallas_call(
        _copy_kernel,
        grid=(m // blk,),
        in_specs=[pl.BlockSpec((blk, n), lambda i: (i, 0))],
        out_specs=pl.BlockSpec((blk, n), lambda i: (i, 0)),
        out_shape=jax.ShapeDtypeStruct((m, n), x.dtype),
        compiler_params=pltpu.CompilerParams(
            dimension_semantics=("parallel",),
        ),
    )(x)
    return out


# speedup vs baseline: 46.4552x; 1.5799x over previous
import jax
import jax.numpy as jnp
from jax.experimental import pallas as pl
from jax.experimental.pallas import tpu as pltpu

_CHUNKS = 8


def _dma_copy_kernel(x_ref, o_ref, bufs, in_sems, out_sems):
    rows = x_ref.shape[0] // _CHUNKS

    def in_copy(c):
        return pltpu.make_async_copy(
            x_ref.at[pl.ds(c * rows, rows), :],
            bufs.at[c],
            in_sems.at[c],
        )

    def out_copy(c):
        return pltpu.make_async_copy(
            bufs.at[c],
            o_ref.at[pl.ds(c * rows, rows), :],
            out_sems.at[c],
        )

    for c in range(_CHUNKS):
        in_copy(c).start()
    for c in range(_CHUNKS):
        in_copy(c).wait()
        out_copy(c).start()
    for c in range(_CHUNKS):
        out_copy(c).wait()


def kernel(x, scale_p, scale_n):
    del scale_p, scale_n
    m, n = x.shape
    rows = m // _CHUNKS
    out = pl.pallas_call(
        _dma_copy_kernel,
        in_specs=[pl.BlockSpec(memory_space=pl.ANY)],
        out_specs=pl.BlockSpec(memory_space=pl.ANY),
        out_shape=jax.ShapeDtypeStruct((m, n), x.dtype),
        scratch_shapes=[
            pltpu.VMEM((_CHUNKS, rows, n), x.dtype),
            pltpu.SemaphoreType.DMA((_CHUNKS,)),
            pltpu.SemaphoreType.DMA((_CHUNKS,)),
        ],
    )(x)
    return out
